# baseline (speedup 1.0000x reference)
import functools

import jax
import jax.numpy as jnp
import numpy as np
from jax import lax
from jax.experimental import pallas as pl
from jax.experimental.pallas import tpu as pltpu
from jax.experimental.pallas import tpu_sc as plsc

_ALPHA = 0.9
_MCL = 10
_L = 809
_D = 128
_B = 4
_LP = 832
_NC = (_L - _MCL) * _MCL + _MCL * (_MCL + 1) // 2
_OW = 5 * _D
_NI = 8
_ROWS = _NI * _MCL
_WROWS = 27
_NCHUNK = 99
_SP_I0 = 792
_SP_R0 = _SP_I0 * _MCL
_SP_ROWS = _NC - _SP_R0
_NW = 32
_SPLITS = (0, 50, _NCHUNK)


@functools.lru_cache(maxsize=None)
def _scan_mats():
    t_idx = np.arange(_LP)[:, None]
    l_idx = np.arange(_LP)[None, :]
    valid = l_idx < _L
    tf = np.where((l_idx <= t_idx - 1) & valid & (t_idx <= _L),
                  _ALPHA ** np.maximum(t_idx - 1 - l_idx, 0), 0.0)
    tb = np.where((l_idx >= t_idx) & valid & (t_idx < _L),
                  _ALPHA ** np.maximum(l_idx - t_idx, 0), 0.0)
    return jnp.asarray(tf, jnp.float32), jnp.asarray(tb, jnp.float32)


def _scan_body(xp_ref, tf_ref, tb_ref, fp_ref, bk_ref):
    x = xp_ref[0]
    fp_ref[0] = jax.lax.dot(tf_ref[...], x,
                            preferred_element_type=jnp.float32)
    bk_ref[0] = jax.lax.dot(tb_ref[...], x,
                            preferred_element_type=jnp.float32)


def _emit_row(fpw, bkw, obuf, row, wi, wj, s, scale):
    fb = fpw[pl.ds(wi * _D + 16 * s, 16)]
    bb = bkw[pl.ds(wi * _D + 16 * s, 16)]
    fj = fpw[pl.ds(wj * _D + 16 * s, 16)]
    bj = bkw[pl.ds(wj * _D + 16 * s, 16)]
    obuf[row, pl.ds(16 * s, 16)] = fj - scale * fb
    obuf[row, pl.ds(_D + 16 * s, 16)] = fb
    obuf[row, pl.ds(2 * _D + 16 * s, 16)] = fj
    obuf[row, pl.ds(3 * _D + 16 * s, 16)] = bj
    obuf[row, pl.ds(4 * _D + 16 * s, 16)] = bb


def _assemble_main(fpw, bkw, obuf, n_start):
    def body(i_l, carry):
        for s in range(_D // 16):
            for r in range(_MCL):
                _emit_row(fpw, bkw, obuf, i_l * _MCL + r,
                          i_l, i_l + r + 1, s,
                          np.float32(_ALPHA ** (r + 1)))
        return carry

    lax.fori_loop(0, n_start, body, 0)


def _assemble_special(fpw, bkw, obuf):
    _assemble_main(fpw, bkw, obuf, _L - _MCL - _SP_I0)

    def body(s, carry):
        row = (_L - _MCL - _SP_I0) * _MCL
        for it in range(_MCL):
            for r in range(_MCL - it):
                wi = _L - _MCL + it - _SP_I0
                _emit_row(fpw, bkw, obuf, row, wi, wi + r + 1, s,
                          np.float32(_ALPHA ** (r + 1)))
                row += 1
        return carry

    lax.fori_loop(0, _D // 16, body, 0)


def _sc_expand_body(lo, hi, special, fp_hbm, bk_hbm, out_hbm,
                    fpw, bkw, obuf):
    wid = lax.axis_index("s") * 2 + lax.axis_index("c")
    nchunk = hi - lo
    row0 = lo * _NI * _MCL

    def main_task(q, carry):
        g = q * _NW + wid

        @pl.when(g < _B * nchunk)
        def _():
            b = g // nchunk
            k = lo + g - b * nchunk
            i0 = _NI * k
            src = (b * _LP + i0) * _D
            pltpu.sync_copy(fp_hbm.at[pl.ds(src, _WROWS * _D)], fpw)
            pltpu.sync_copy(bk_hbm.at[pl.ds(src, _WROWS * _D)], bkw)
            _assemble_main(fpw, bkw, obuf, _NI)
            pltpu.sync_copy(
                obuf.at[pl.ds(0, _ROWS), :],
                out_hbm.at[b, pl.ds(i0 * _MCL - row0, _ROWS), :])
        return carry

    lax.fori_loop(0, (_B * nchunk + _NW - 1) // _NW, main_task, 0)

    if special:
        @pl.when(wid < _B)
        def _():
            b = wid
            src = (b * _LP + _SP_I0) * _D
            pltpu.sync_copy(fp_hbm.at[pl.ds(src, _WROWS * _D)], fpw)
            pltpu.sync_copy(bk_hbm.at[pl.ds(src, _WROWS * _D)], bkw)
            _assemble_special(fpw, bkw, obuf)
            pltpu.sync_copy(
                obuf.at[pl.ds(0, _SP_ROWS), :],
                out_hbm.at[b, pl.ds(_SP_R0 - row0, _SP_ROWS), :])


def kernel(x_input, x_mask):
    del x_mask
    tf, tb = _scan_mats()
    xp = jnp.pad(x_input, ((0, 0), (0, _LP - _L), (0, 0)))

    fp, bk = pl.pallas_call(
        _scan_body,
        grid=(_B,),
        in_specs=[
            pl.BlockSpec((1, _LP, _D), lambda b: (b, 0, 0)),
            pl.BlockSpec((_LP, _LP), lambda b: (0, 0)),
            pl.BlockSpec((_LP, _LP), lambda b: (0, 0)),
        ],
        out_specs=[
            pl.BlockSpec((1, _LP, _D), lambda b: (b, 0, 0)),
            pl.BlockSpec((1, _LP, _D), lambda b: (b, 0, 0)),
        ],
        out_shape=[
            jax.ShapeDtypeStruct((_B, _LP, _D), jnp.float32),
            jax.ShapeDtypeStruct((_B, _LP, _D), jnp.float32),
        ],
    )(xp, tf, tb)

    mesh = plsc.VectorSubcoreMesh(core_axis_name="c", subcore_axis_name="s")
    fp1, bk1 = fp.reshape(-1), bk.reshape(-1)
    parts = []
    for p in range(len(_SPLITS) - 1):
        lo, hi = _SPLITS[p], _SPLITS[p + 1]
        special = hi == _NCHUNK
        nrows = (hi * _NI * _MCL - lo * _NI * _MCL
                 + (_SP_ROWS - (_NCHUNK * _NI - _SP_I0) * _MCL
                    if special else 0))
        part = functools.partial(
            pl.kernel,
            mesh=mesh,
            out_type=jax.ShapeDtypeStruct((_B, nrows, _OW), jnp.float32),
            scratch_types=[
                pltpu.VMEM((_WROWS * _D,), jnp.float32),
                pltpu.VMEM((_WROWS * _D,), jnp.float32),
                pltpu.VMEM((_SP_ROWS, _OW), jnp.float32),
            ],
        )(functools.partial(_sc_expand_body, lo, hi, special))
        parts.append(part(fp1, bk1))
    return jnp.concatenate(parts, axis=1)


# speedup vs baseline: 2.1925x; 2.1925x over previous
import functools

import jax
import jax.numpy as jnp
import numpy as np
from jax import lax
from jax.experimental import pallas as pl
from jax.experimental.pallas import tpu as pltpu
from jax.experimental.pallas import tpu_sc as plsc

_ALPHA = 0.9
_MCL = 10
_L = 809
_D = 128
_B = 4
_LP = 832
_NC = (_L - _MCL) * _MCL + _MCL * (_MCL + 1) // 2
_OW = 5 * _D
_NI = 4
_ROWS = _NI * _MCL
_WROWS = 18
_NCHUNK = 198
_SA_I0 = 792
_SA_R0 = _SA_I0 * _MCL
_SB_I0 = 796
_SB_R0 = _SB_I0 * _MCL
_SC_I0 = 800
_SC_R0 = 8000
_SC_ROWS = _NC - _SC_R0
_NW = 32


@functools.lru_cache(maxsize=None)
def _scan_mats():
    t_idx = np.arange(_LP)[:, None]
    l_idx = np.arange(_LP)[None, :]
    valid = l_idx < _L
    tf = np.where((l_idx <= t_idx - 1) & valid & (t_idx <= _L),
                  _ALPHA ** np.maximum(t_idx - 1 - l_idx, 0), 0.0)
    tb = np.where((l_idx >= t_idx) & valid & (t_idx < _L),
                  _ALPHA ** np.maximum(l_idx - t_idx, 0), 0.0)
    return jnp.asarray(tf, jnp.float32), jnp.asarray(tb, jnp.float32)


def _scan_body(xp_ref, tf_ref, tb_ref, fp_ref, bk_ref):
    x = xp_ref[0]
    fp_ref[0] = jax.lax.dot(tf_ref[...], x,
                            preferred_element_type=jnp.float32)
    bk_ref[0] = jax.lax.dot(tb_ref[...], x,
                            preferred_element_type=jnp.float32)


def _emit_row(fpw, bkw, obuf, row, wi, wj, s, scale):
    fb = fpw[pl.ds(wi * _D + 16 * s, 16)]
    bb = bkw[pl.ds(wi * _D + 16 * s, 16)]
    fj = fpw[pl.ds(wj * _D + 16 * s, 16)]
    bj = bkw[pl.ds(wj * _D + 16 * s, 16)]
    obuf[row, pl.ds(16 * s, 16)] = fj - scale * fb
    obuf[row, pl.ds(_D + 16 * s, 16)] = fb
    obuf[row, pl.ds(2 * _D + 16 * s, 16)] = fj
    obuf[row, pl.ds(3 * _D + 16 * s, 16)] = bj
    obuf[row, pl.ds(4 * _D + 16 * s, 16)] = bb


def _fetch_windows(fp_hbm, bk_hbm, fpw, bkw, b, i0):
    src = (b * _LP + i0) * _D
    pltpu.sync_copy(fp_hbm.at[pl.ds(src, _WROWS * _D)], fpw)
    pltpu.sync_copy(bk_hbm.at[pl.ds(src, _WROWS * _D)], bkw)


def _assemble_main(fpw, bkw, obuf, n_start):
    def body(i_l, carry):
        for s in range(_D // 16):
            for r in range(_MCL):
                _emit_row(fpw, bkw, obuf, i_l * _MCL + r,
                          i_l, i_l + r + 1, s,
                          np.float32(_ALPHA ** (r + 1)))
        return carry

    lax.fori_loop(0, n_start, body, 0)


def _assemble_tail(fpw, bkw, obuf, row0, it_lo, it_hi, base_i):
    def body(s, carry):
        row = row0
        for it in range(it_lo, it_hi):
            wi = _L - _MCL + it - base_i
            for r in range(_MCL - it):
                _emit_row(fpw, bkw, obuf, row, wi, wi + r + 1, s,
                          np.float32(_ALPHA ** (r + 1)))
                row += 1
        return carry

    lax.fori_loop(0, _D // 16, body, 0)


def _sc_expand_body(fp_hbm, bk_hbm, out_hbm,
                    fpw, bkw, obuf0, obuf1, obufb, sem0, sem1):
    wid = lax.axis_index("s") * 2 + lax.axis_index("c")
    bufs = ((obuf0, sem0), (obuf1, sem1))

    def run_main(g, obuf, sem):
        b = g // _NCHUNK
        i0 = _NI * (g - b * _NCHUNK)
        _fetch_windows(fp_hbm, bk_hbm, fpw, bkw, b, i0)
        _assemble_main(fpw, bkw, obuf, _NI)
        pltpu.async_copy(obuf,
                         out_hbm.at[b, pl.ds(i0 * _MCL, _ROWS), :], sem)

    def pair(qq, carry):
        for par in range(2):
            g = (qq * 2 + par) * _NW + wid
            obuf, sem = bufs[par]

            @pl.when(g < _B * _NCHUNK)
            def _():
                run_main(g, obuf, sem)
        for par in range(2):
            g = (qq * 2 + par) * _NW + wid
            obuf, sem = bufs[par]

            @pl.when(g < _B * _NCHUNK)
            def _():
                pltpu.make_async_copy(
                    obuf, out_hbm.at[0, pl.ds(0, _ROWS), :], sem).wait()
        return carry

    npair = (_B * _NCHUNK + 2 * _NW - 1) // (2 * _NW)
    lax.fori_loop(0, npair, pair, 0)

    @pl.when(wid < _B)
    def _():
        b = wid
        _fetch_windows(fp_hbm, bk_hbm, fpw, bkw, b, _SA_I0)
        _assemble_main(fpw, bkw, obuf0, _NI)
        pltpu.sync_copy(obuf0, out_hbm.at[b, pl.ds(_SA_R0, _ROWS), :])

    @pl.when((wid >= _B) & (wid < 2 * _B))
    def _():
        b = wid - _B
        _fetch_windows(fp_hbm, bk_hbm, fpw, bkw, b, _SB_I0)
        _assemble_main(fpw, bkw, obuf0, _L - _MCL - _SB_I0)
        _assemble_tail(fpw, bkw, obuf0,
                       (_L - _MCL - _SB_I0) * _MCL, 0, 1, _SB_I0)
        pltpu.sync_copy(obuf0, out_hbm.at[b, pl.ds(_SB_R0, _ROWS), :])

    @pl.when((wid >= 2 * _B) & (wid < 3 * _B))
    def _():
        b = wid - 2 * _B
        _fetch_windows(fp_hbm, bk_hbm, fpw, bkw, b, _SC_I0)
        _assemble_tail(fpw, bkw, obufb, 0, 1, _MCL, _SC_I0)
        pltpu.sync_copy(obufb, out_hbm.at[b, pl.ds(_SC_R0, _SC_ROWS), :])


def kernel(x_input, x_mask):
    del x_mask
    tf, tb = _scan_mats()
    xp = jnp.pad(x_input, ((0, 0), (0, _LP - _L), (0, 0)))

    fp, bk = pl.pallas_call(
        _scan_body,
        grid=(_B,),
        in_specs=[
            pl.BlockSpec((1, _LP, _D), lambda b: (b, 0, 0)),
            pl.BlockSpec((_LP, _LP), lambda b: (0, 0)),
            pl.BlockSpec((_LP, _LP), lambda b: (0, 0)),
        ],
        out_specs=[
            pl.BlockSpec((1, _LP, _D), lambda b: (b, 0, 0)),
            pl.BlockSpec((1, _LP, _D), lambda b: (b, 0, 0)),
        ],
        out_shape=[
            jax.ShapeDtypeStruct((_B, _LP, _D), jnp.float32),
            jax.ShapeDtypeStruct((_B, _LP, _D), jnp.float32),
        ],
    )(xp, tf, tb)

    mesh = plsc.VectorSubcoreMesh(core_axis_name="c", subcore_axis_name="s")
    expand = functools.partial(
        pl.kernel,
        mesh=mesh,
        out_type=jax.ShapeDtypeStruct((_B, _NC, _OW), jnp.float32),
        scratch_types=[
            pltpu.VMEM((_WROWS * _D,), jnp.float32),
            pltpu.VMEM((_WROWS * _D,), jnp.float32),
            pltpu.VMEM((_ROWS, _OW), jnp.float32),
            pltpu.VMEM((_ROWS, _OW), jnp.float32),
            pltpu.VMEM((_SC_ROWS, _OW), jnp.float32),
            pltpu.SemaphoreType.DMA,
            pltpu.SemaphoreType.DMA,
        ],
    )(_sc_expand_body)
    return expand(fp.reshape(-1), bk.reshape(-1))


# speedup vs baseline: 2.3435x; 1.0689x over previous
import functools

import jax
import jax.numpy as jnp
import numpy as np
from jax import lax
from jax.experimental import pallas as pl
from jax.experimental.pallas import tpu as pltpu
from jax.experimental.pallas import tpu_sc as plsc

_ALPHA = 0.9
_MCL = 10
_L = 809
_D = 128
_B = 4
_LP = 832
_NC = (_L - _MCL) * _MCL + _MCL * (_MCL + 1) // 2
_OW = 5 * _D
_NI = 8
_ROWS = _NI * _MCL
_WROWS = 27
_NCHUNK = 99
_SP_I0 = 792
_SP_R0 = _SP_I0 * _MCL
_SP_ROWS = _NC - _SP_R0
_NW = 32


@functools.lru_cache(maxsize=None)
def _scan_mats():
    t_idx = np.arange(_LP)[:, None]
    l_idx = np.arange(_LP)[None, :]
    valid = l_idx < _L
    tf = np.where((l_idx <= t_idx - 1) & valid & (t_idx <= _L),
                  _ALPHA ** np.maximum(t_idx - 1 - l_idx, 0), 0.0)
    tb = np.where((l_idx >= t_idx) & valid & (t_idx < _L),
                  _ALPHA ** np.maximum(l_idx - t_idx, 0), 0.0)
    return jnp.asarray(tf, jnp.float32), jnp.asarray(tb, jnp.float32)


def _scan_body(xp_ref, tf_ref, tb_ref, fp_ref, bk_ref):
    x = xp_ref[0]
    fp_ref[0] = jax.lax.dot(tf_ref[...], x,
                            preferred_element_type=jnp.float32)
    bk_ref[0] = jax.lax.dot(tb_ref[...], x,
                            preferred_element_type=jnp.float32)


def _emit_row(fpw, bkw, obuf, row, wi, wj, s, scale):
    fb = fpw[pl.ds(wi * _D + 16 * s, 16)]
    bb = bkw[pl.ds(wi * _D + 16 * s, 16)]
    fj = fpw[pl.ds(wj * _D + 16 * s, 16)]
    bj = bkw[pl.ds(wj * _D + 16 * s, 16)]
    obuf[row, pl.ds(16 * s, 16)] = fj - scale * fb
    obuf[row, pl.ds(_D + 16 * s, 16)] = fb
    obuf[row, pl.ds(2 * _D + 16 * s, 16)] = fj
    obuf[row, pl.ds(3 * _D + 16 * s, 16)] = bj
    obuf[row, pl.ds(4 * _D + 16 * s, 16)] = bb


def _fetch_windows(fp_hbm, bk_hbm, fpw, bkw, sem0, sem1, b, i0):
    src = (b * _LP + i0) * _D
    cf = pltpu.async_copy(fp_hbm.at[pl.ds(src, _WROWS * _D)], fpw, sem0)
    cb = pltpu.async_copy(bk_hbm.at[pl.ds(src, _WROWS * _D)], bkw, sem1)
    cf.wait()
    cb.wait()


def _assemble_main(fpw, bkw, obuf, n_start):
    def body(i_l, carry):
        for s in range(_D // 16):
            for r in range(_MCL):
                _emit_row(fpw, bkw, obuf, i_l * _MCL + r,
                          i_l, i_l + r + 1, s,
                          np.float32(_ALPHA ** (r + 1)))
        return carry

    lax.fori_loop(0, n_start, body, 0)


def _assemble_special(fpw, bkw, obuf):
    _assemble_main(fpw, bkw, obuf, _L - _MCL - _SP_I0)

    def body(s, carry):
        row = (_L - _MCL - _SP_I0) * _MCL
        for it in range(_MCL):
            wi = _L - _MCL + it - _SP_I0
            for r in range(_MCL - it):
                _emit_row(fpw, bkw, obuf, row, wi, wi + r + 1, s,
                          np.float32(_ALPHA ** (r + 1)))
                row += 1
        return carry

    lax.fori_loop(0, _D // 16, body, 0)


def _sc_expand_body(fp_hbm, bk_hbm, out_hbm, fpw, bkw, obuf, sem0, sem1):
    wid = lax.axis_index("s") * 2 + lax.axis_index("c")

    def main_task(q, carry):
        g = q * _NW + wid

        @pl.when(g < _B * _NCHUNK)
        def _():
            b = g // _NCHUNK
            i0 = _NI * (g - b * _NCHUNK)
            _fetch_windows(fp_hbm, bk_hbm, fpw, bkw, sem0, sem1, b, i0)
            _assemble_main(fpw, bkw, obuf, _NI)
            pltpu.sync_copy(
                obuf.at[pl.ds(0, _ROWS), :],
                out_hbm.at[b, pl.ds(i0 * _MCL, _ROWS), :])
        return carry

    lax.fori_loop(0, (_B * _NCHUNK + _NW - 1) // _NW, main_task, 0)

    @pl.when(wid < _B)
    def _():
        b = wid
        _fetch_windows(fp_hbm, bk_hbm, fpw, bkw, sem0, sem1, b, _SP_I0)
        _assemble_special(fpw, bkw, obuf)
        pltpu.sync_copy(obuf,
                        out_hbm.at[b, pl.ds(_SP_R0, _SP_ROWS), :])


def kernel(x_input, x_mask):
    del x_mask
    tf, tb = _scan_mats()
    xp = jnp.pad(x_input, ((0, 0), (0, _LP - _L), (0, 0)))

    fp, bk = pl.pallas_call(
        _scan_body,
        grid=(_B,),
        in_specs=[
            pl.BlockSpec((1, _LP, _D), lambda b: (b, 0, 0)),
            pl.BlockSpec((_LP, _LP), lambda b: (0, 0)),
            pl.BlockSpec((_LP, _LP), lambda b: (0, 0)),
        ],
        out_specs=[
            pl.BlockSpec((1, _LP, _D), lambda b: (b, 0, 0)),
            pl.BlockSpec((1, _LP, _D), lambda b: (b, 0, 0)),
        ],
        out_shape=[
            jax.ShapeDtypeStruct((_B, _LP, _D), jnp.float32),
            jax.ShapeDtypeStruct((_B, _LP, _D), jnp.float32),
        ],
    )(xp, tf, tb)

    mesh = plsc.VectorSubcoreMesh(core_axis_name="c", subcore_axis_name="s")
    expand = functools.partial(
        pl.kernel,
        mesh=mesh,
        out_type=jax.ShapeDtypeStruct((_B, _NC, _OW), jnp.float32),
        scratch_types=[
            pltpu.VMEM((_WROWS * _D,), jnp.float32),
            pltpu.VMEM((_WROWS * _D,), jnp.float32),
            pltpu.VMEM((_SP_ROWS, _OW), jnp.float32),
            pltpu.SemaphoreType.DMA,
            pltpu.SemaphoreType.DMA,
        ],
    )(_sc_expand_body)
    return expand(fp.reshape(-1), bk.reshape(-1))


# speedup vs baseline: 2.4296x; 1.0367x over previous
import functools

import jax
import jax.numpy as jnp
import numpy as np
from jax import lax
from jax.experimental import pallas as pl
from jax.experimental.pallas import tpu as pltpu
from jax.experimental.pallas import tpu_sc as plsc

_ALPHA = 0.9
_MCL = 10
_L = 809
_D = 128
_B = 4
_LP = 832
_NC = (_L - _MCL) * _MCL + _MCL * (_MCL + 1) // 2
_OW = 5 * _D
_NI = 8
_ROWS = _NI * _MCL
_WROWS = 18
_NCHUNK = 99
_SP_I0 = 792
_SP_R0 = _SP_I0 * _MCL
_SP_ROWS = _NC - _SP_R0
_NW = 32


@functools.lru_cache(maxsize=None)
def _scan_mats():
    t_idx = np.arange(_LP)[:, None]
    l_idx = np.arange(_LP)[None, :]
    valid = l_idx < _L
    tf = np.where((l_idx <= t_idx - 1) & valid & (t_idx <= _L),
                  _ALPHA ** np.maximum(t_idx - 1 - l_idx, 0), 0.0)
    tb = np.where((l_idx >= t_idx) & valid & (t_idx < _L),
                  _ALPHA ** np.maximum(l_idx - t_idx, 0), 0.0)
    return jnp.asarray(tf, jnp.float32), jnp.asarray(tb, jnp.float32)


def _scan_body(xp_ref, tf_ref, tb_ref, fp_ref, bk_ref):
    x = xp_ref[0]
    fp_ref[0] = jax.lax.dot(tf_ref[...], x,
                            preferred_element_type=jnp.float32)
    bk_ref[0] = jax.lax.dot(tb_ref[...], x,
                            preferred_element_type=jnp.float32)


def _emit_row(fpw, bkw, obuf, row, wi, wj, s, scale):
    fb = fpw[pl.ds(wi * _D + 16 * s, 16)]
    bb = bkw[pl.ds(wi * _D + 16 * s, 16)]
    fj = fpw[pl.ds(wj * _D + 16 * s, 16)]
    bj = bkw[pl.ds(wj * _D + 16 * s, 16)]
    obuf[row, pl.ds(16 * s, 16)] = fj - scale * fb
    obuf[row, pl.ds(_D + 16 * s, 16)] = fb
    obuf[row, pl.ds(2 * _D + 16 * s, 16)] = fj
    obuf[row, pl.ds(3 * _D + 16 * s, 16)] = bj
    obuf[row, pl.ds(4 * _D + 16 * s, 16)] = bb


def _fetch_windows(fp_hbm, bk_hbm, fpw, bkw, sem0, sem1, b, i0):
    src = (b * _LP + i0) * _D
    cf = pltpu.async_copy(fp_hbm.at[pl.ds(src, _WROWS * _D)], fpw, sem0)
    cb = pltpu.async_copy(bk_hbm.at[pl.ds(src, _WROWS * _D)], bkw, sem1)
    cf.wait()
    cb.wait()


def _assemble_main(fpw, bkw, obuf, n_start):
    def body(i_l, carry):
        for s in range(_D // 16):
            for r in range(_MCL):
                _emit_row(fpw, bkw, obuf, i_l * _MCL + r,
                          i_l, i_l + r + 1, s,
                          np.float32(_ALPHA ** (r + 1)))
        return carry

    lax.fori_loop(0, n_start, body, 0)


def _assemble_special(fpw, bkw, obuf):
    _assemble_main(fpw, bkw, obuf, _L - _MCL - _SP_I0)

    def body(s, carry):
        row = (_L - _MCL - _SP_I0) * _MCL
        for it in range(_MCL):
            wi = _L - _MCL + it - _SP_I0
            for r in range(_MCL - it):
                _emit_row(fpw, bkw, obuf, row, wi, wi + r + 1, s,
                          np.float32(_ALPHA ** (r + 1)))
                row += 1
        return carry

    lax.fori_loop(0, _D // 16, body, 0)


def _sc_expand_body(fp_hbm, bk_hbm, out_hbm, fpw, bkw, obuf, sem0, sem1):
    wid = lax.axis_index("s") * 2 + lax.axis_index("c")

    def main_task(q, carry):
        g = q * _NW + wid

        @pl.when(g < _B * _NCHUNK)
        def _():
            b = g // _NCHUNK
            i0 = _NI * (g - b * _NCHUNK)
            _fetch_windows(fp_hbm, bk_hbm, fpw, bkw, sem0, sem1, b, i0)
            _assemble_main(fpw, bkw, obuf, _NI)
            pltpu.sync_copy(
                obuf.at[pl.ds(0, _ROWS), :],
                out_hbm.at[b, pl.ds(i0 * _MCL, _ROWS), :])
        return carry

    lax.fori_loop(0, (_B * _NCHUNK + _NW - 1) // _NW, main_task, 0)

    @pl.when(wid >= _NW - _B)
    def _():
        b = wid - (_NW - _B)
        _fetch_windows(fp_hbm, bk_hbm, fpw, bkw, sem0, sem1, b, _SP_I0)
        _assemble_special(fpw, bkw, obuf)
        pltpu.sync_copy(obuf,
                        out_hbm.at[b, pl.ds(_SP_R0, _SP_ROWS), :])


def kernel(x_input, x_mask):
    del x_mask
    tf, tb = _scan_mats()
    xp = jnp.pad(x_input, ((0, 0), (0, _LP - _L), (0, 0)))

    fp, bk = pl.pallas_call(
        _scan_body,
        grid=(_B,),
        in_specs=[
            pl.BlockSpec((1, _LP, _D), lambda b: (b, 0, 0)),
            pl.BlockSpec((_LP, _LP), lambda b: (0, 0)),
            pl.BlockSpec((_LP, _LP), lambda b: (0, 0)),
        ],
        out_specs=[
            pl.BlockSpec((1, _LP, _D), lambda b: (b, 0, 0)),
            pl.BlockSpec((1, _LP, _D), lambda b: (b, 0, 0)),
        ],
        out_shape=[
            jax.ShapeDtypeStruct((_B, _LP, _D), jnp.float32),
            jax.ShapeDtypeStruct((_B, _LP, _D), jnp.float32),
        ],
    )(xp, tf, tb)

    mesh = plsc.VectorSubcoreMesh(core_axis_name="c", subcore_axis_name="s")
    expand = functools.partial(
        pl.kernel,
        mesh=mesh,
        out_type=jax.ShapeDtypeStruct((_B, _NC, _OW), jnp.float32),
        scratch_types=[
            pltpu.VMEM((_WROWS * _D,), jnp.float32),
            pltpu.VMEM((_WROWS * _D,), jnp.float32),
            pltpu.VMEM((_SP_ROWS, _OW), jnp.float32),
            pltpu.SemaphoreType.DMA,
            pltpu.SemaphoreType.DMA,
        ],
    )(_sc_expand_body)
    return expand(fp.reshape(-1), bk.reshape(-1))


# speedup vs baseline: 2.4741x; 1.0183x over previous
import functools

import jax
import jax.numpy as jnp
import numpy as np
from jax import lax
from jax.experimental import pallas as pl
from jax.experimental.pallas import tpu as pltpu
from jax.experimental.pallas import tpu_sc as plsc

_ALPHA = 0.9
_MCL = 10
_L = 809
_D = 128
_B = 4
_LP = 832
_NC = (_L - _MCL) * _MCL + _MCL * (_MCL + 1) // 2
_OW = 5 * _D
_NI = 12
_ROWS = _NI * _MCL
_WROWS = 22
_NCHUNK = 66
_SP_I0 = 792
_SP_R0 = _SP_I0 * _MCL
_SP_ROWS = _NC - _SP_R0
_NW = 32


@functools.lru_cache(maxsize=None)
def _scan_mats():
    t_idx = np.arange(_LP)[:, None]
    l_idx = np.arange(_LP)[None, :]
    valid = l_idx < _L
    tf = np.where((l_idx <= t_idx - 1) & valid & (t_idx <= _L),
                  _ALPHA ** np.maximum(t_idx - 1 - l_idx, 0), 0.0)
    tb = np.where((l_idx >= t_idx) & valid & (t_idx < _L),
                  _ALPHA ** np.maximum(l_idx - t_idx, 0), 0.0)
    return jnp.asarray(tf, jnp.float32), jnp.asarray(tb, jnp.float32)


def _scan_body(xp_ref, tf_ref, tb_ref, fp_ref, bk_ref):
    x = xp_ref[0]
    fp_ref[0] = jax.lax.dot(tf_ref[...], x,
                            preferred_element_type=jnp.float32)
    bk_ref[0] = jax.lax.dot(tb_ref[...], x,
                            preferred_element_type=jnp.float32)


def _emit_row(fpw, bkw, obuf, row, wi, wj, s, scale):
    fb = fpw[pl.ds(wi * _D + 16 * s, 16)]
    bb = bkw[pl.ds(wi * _D + 16 * s, 16)]
    fj = fpw[pl.ds(wj * _D + 16 * s, 16)]
    bj = bkw[pl.ds(wj * _D + 16 * s, 16)]
    obuf[row, pl.ds(16 * s, 16)] = fj - scale * fb
    obuf[row, pl.ds(_D + 16 * s, 16)] = fb
    obuf[row, pl.ds(2 * _D + 16 * s, 16)] = fj
    obuf[row, pl.ds(3 * _D + 16 * s, 16)] = bj
    obuf[row, pl.ds(4 * _D + 16 * s, 16)] = bb


def _fetch_windows(fp_hbm, bk_hbm, fpw, bkw, sem0, sem1, b, i0):
    src = (b * _LP + i0) * _D
    cf = pltpu.async_copy(fp_hbm.at[pl.ds(src, _WROWS * _D)], fpw, sem0)
    cb = pltpu.async_copy(bk_hbm.at[pl.ds(src, _WROWS * _D)], bkw, sem1)
    cf.wait()
    cb.wait()


def _assemble_main(fpw, bkw, obuf, n_start):
    def body(i_l, carry):
        for s in range(_D // 16):
            for r in range(_MCL):
                _emit_row(fpw, bkw, obuf, i_l * _MCL + r,
                          i_l, i_l + r + 1, s,
                          np.float32(_ALPHA ** (r + 1)))
        return carry

    lax.fori_loop(0, n_start, body, 0)


def _assemble_special(fpw, bkw, obuf):
    _assemble_main(fpw, bkw, obuf, _L - _MCL - _SP_I0)

    def body(s, carry):
        row = (_L - _MCL - _SP_I0) * _MCL
        for it in range(_MCL):
            wi = _L - _MCL + it - _SP_I0
            for r in range(_MCL - it):
                _emit_row(fpw, bkw, obuf, row, wi, wi + r + 1, s,
                          np.float32(_ALPHA ** (r + 1)))
                row += 1
        return carry

    lax.fori_loop(0, _D // 16, body, 0)


def _sc_expand_body(fp_hbm, bk_hbm, out_hbm, fpw, bkw, obuf, sem0, sem1):
    wid = lax.axis_index("s") * 2 + lax.axis_index("c")

    def main_task(q, carry):
        g = q * _NW + wid

        @pl.when(g < _B * _NCHUNK)
        def _():
            b = g // _NCHUNK
            i0 = _NI * (g - b * _NCHUNK)
            _fetch_windows(fp_hbm, bk_hbm, fpw, bkw, sem0, sem1, b, i0)
            _assemble_main(fpw, bkw, obuf, _NI)
            pltpu.sync_copy(
                obuf.at[pl.ds(0, _ROWS), :],
                out_hbm.at[b, pl.ds(i0 * _MCL, _ROWS), :])
        return carry

    lax.fori_loop(0, (_B * _NCHUNK + _NW - 1) // _NW, main_task, 0)

    @pl.when(wid >= _NW - _B)
    def _():
        b = wid - (_NW - _B)
        _fetch_windows(fp_hbm, bk_hbm, fpw, bkw, sem0, sem1, b, _SP_I0)
        _assemble_special(fpw, bkw, obuf)
        pltpu.sync_copy(obuf,
                        out_hbm.at[b, pl.ds(_SP_R0, _SP_ROWS), :])


def kernel(x_input, x_mask):
    del x_mask
    tf, tb = _scan_mats()
    xp = jnp.pad(x_input, ((0, 0), (0, _LP - _L), (0, 0)))

    fp, bk = pl.pallas_call(
        _scan_body,
        grid=(_B,),
        in_specs=[
            pl.BlockSpec((1, _LP, _D), lambda b: (b, 0, 0)),
            pl.BlockSpec((_LP, _LP), lambda b: (0, 0)),
            pl.BlockSpec((_LP, _LP), lambda b: (0, 0)),
        ],
        out_specs=[
            pl.BlockSpec((1, _LP, _D), lambda b: (b, 0, 0)),
            pl.BlockSpec((1, _LP, _D), lambda b: (b, 0, 0)),
        ],
        out_shape=[
            jax.ShapeDtypeStruct((_B, _LP, _D), jnp.float32),
            jax.ShapeDtypeStruct((_B, _LP, _D), jnp.float32),
        ],
    )(xp, tf, tb)

    mesh = plsc.VectorSubcoreMesh(core_axis_name="c", subcore_axis_name="s")
    expand = functools.partial(
        pl.kernel,
        mesh=mesh,
        out_type=jax.ShapeDtypeStruct((_B, _NC, _OW), jnp.float32),
        scratch_types=[
            pltpu.VMEM((_WROWS * _D,), jnp.float32),
            pltpu.VMEM((_WROWS * _D,), jnp.float32),
            pltpu.VMEM((_SP_ROWS, _OW), jnp.float32),
            pltpu.SemaphoreType.DMA,
            pltpu.SemaphoreType.DMA,
        ],
    )(_sc_expand_body)
    return expand(fp.reshape(-1), bk.reshape(-1))
